# Initial kernel scaffold; baseline (speedup 1.0000x reference)
import jax
import jax.numpy as jnp
from jax.experimental import pallas as pl


def kernel(mesh_pos, node_type, u, load, senders, receivers, ne_W1, ne_b1, ne_W2, ne_b2, ne_g, ne_bb, ee_W1, ee_b1, ee_W2, ee_b2, ee_g, ee_bb, be_W1, be_b1, be_W2, be_b2, be_g, be_bb, bn_W1, bn_b1, bn_W2, bn_b2, bn_g, bn_bb, dec_W1, dec_b1, dec_W2, dec_b2):
    raise NotImplementedError("write your pallas kernel here")


# speedup vs baseline: 3.0646x; 3.0646x over previous
import functools

import jax
import jax.numpy as jnp
from jax import lax
from jax.experimental import pallas as pl
from jax.experimental.pallas import tpu as pltpu
from jax.experimental.pallas import tpu_sc as plsc

N = 10000
E = 320000
L = 128
STEPS = 5
TW = 5
TD = 3

NWORK = 32
CHUNK = 128
CH_PER_W = 79
E_PAD = NWORK * CH_PER_W * CHUNK
N_PAD = 10240

BLK_E = 2048
BLK_N = 1280

_F32 = jnp.float32


def _ln(h, g, bb):
    mu = jnp.mean(h, axis=-1, keepdims=True)
    var = jnp.mean((h - mu) ** 2, axis=-1, keepdims=True)
    return (h - mu) / jnp.sqrt(var + 1e-5) * g + bb


def _sc_mesh():
    return plsc.VectorSubcoreMesh(core_axis_name="c", subcore_axis_name="s",
                                  num_cores=2, num_subcores=16)


@functools.partial(jax.jit, static_argnames=("D",))
def _sc_gather_add(tabA, tabB, idxS, idxR, D):
    @functools.partial(
        pl.kernel,
        out_type=jax.ShapeDtypeStruct((E_PAD, D), _F32),
        mesh=_sc_mesh(),
        scratch_types=[
            pltpu.VMEM((CH_PER_W, CHUNK), jnp.int32),
            pltpu.VMEM((CH_PER_W, CHUNK), jnp.int32),
            pltpu.VMEM((CHUNK, D), _F32),
            pltpu.VMEM((CHUNK, D), _F32),
            pltpu.SemaphoreType.DMA,
            pltpu.SemaphoreType.DMA,
        ],
    )
    def k(tabA_h, tabB_h, idxS_h, idxR_h, out_h, iS_v, iR_v, bufA, bufB,
          semA, semB):
        wid = lax.axis_index("s") * 2 + lax.axis_index("c")
        pltpu.sync_copy(idxS_h.at[wid], iS_v)
        pltpu.sync_copy(idxR_h.at[wid], iR_v)
        base = wid * (CH_PER_W * CHUNK)

        def chunk(j, carry):
            cpA = pltpu.async_copy(tabA_h.at[iS_v.at[j]], bufA, semA)
            cpB = pltpu.async_copy(tabB_h.at[iR_v.at[j]], bufB, semB)
            cpA.wait()
            cpB.wait()

            def add_row(r, c2):
                for kk in range(D // 16):
                    sl = pl.ds(kk * 16, 16)
                    bufA[r, sl] = bufA[r, sl] + bufB[r, sl]
                return c2

            lax.fori_loop(0, CHUNK, add_row, 0)
            pltpu.sync_copy(bufA, out_h.at[pl.ds(base + j * CHUNK, CHUNK)])
            return carry

        lax.fori_loop(0, CH_PER_W, chunk, 0)

    return k(tabA, tabB, idxS, idxR)


@jax.jit
def _sc_scatter_add(vals, idxR, zrows):
    rows_sub = N_PAD // 16

    @functools.partial(
        pl.kernel,
        out_type=jax.ShapeDtypeStruct((2, N_PAD, L), _F32),
        mesh=_sc_mesh(),
        scratch_types=[
            pltpu.VMEM((CH_PER_W, CHUNK), jnp.int32),
            pltpu.VMEM((CHUNK, L), _F32),
            pltpu.VMEM_SHARED((N_PAD, L), _F32),
            pltpu.SemaphoreType.DMA,
        ],
    )
    def k(vals_h, idxR_h, z_h, out_h, idx_v, buf, acc, sem):
        c = lax.axis_index("c")
        s = lax.axis_index("s")
        wid = s * 2 + c
        pltpu.sync_copy(idxR_h.at[wid], idx_v)
        pltpu.sync_copy(z_h.at[pl.ds(s * rows_sub, rows_sub)],
                        acc.at[pl.ds(s * rows_sub, rows_sub)])
        plsc.subcore_barrier()
        base = wid * (CH_PER_W * CHUNK)

        def chunk(j, carry):
            pltpu.sync_copy(vals_h.at[pl.ds(base + j * CHUNK, CHUNK)], buf)
            pltpu.sync_copy(buf, acc.at[idx_v.at[j]], add=True)
            return carry

        lax.fori_loop(0, CH_PER_W, chunk, 0)
        plsc.subcore_barrier()
        pltpu.sync_copy(acc.at[pl.ds(s * rows_sub, rows_sub)],
                        out_h.at[c, pl.ds(s * rows_sub, rows_sub)])

    return k(vals, idxR, zrows)


def _edge_encode_body(d_ref, W1d, w1n, b1, W2, b2, g, bb, el0):
    d = d_ref[...]
    m = (lax.broadcasted_iota(jnp.int32, (1, L), 1) < 3).astype(_F32)
    norm = jnp.sqrt(jnp.sum((d * m) ** 2, axis=-1, keepdims=True))
    x = jnp.dot(d, W1d[...], preferred_element_type=_F32) + norm * w1n[...] + b1[...]
    h1 = jnp.maximum(x, 0.0)
    h2 = jnp.maximum(jnp.dot(h1, W2[...], preferred_element_type=_F32) + b2[...], 0.0)
    el0[...] = _ln(h2, g[...], bb[...])


def _node_encode_body(nf_ref, W1p, b1, W2, b2, g, bb, WnA, WnB, nl0, A0, B0):
    x = jnp.dot(nf_ref[...], W1p[...], preferred_element_type=_F32) + b1[...]
    h1 = jnp.maximum(x, 0.0)
    h2 = jnp.maximum(jnp.dot(h1, W2[...], preferred_element_type=_F32) + b2[...], 0.0)
    nl = _ln(h2, g[...], bb[...])
    nl0[...] = nl
    A0[...] = jnp.dot(nl, WnA[...], preferred_element_type=_F32)
    B0[...] = jnp.dot(nl, WnB[...], preferred_element_type=_F32)


def _edge_step_body(el_ref, G_ref, W1c, b1, W2, b2, g, bb, elo, nel):
    el = el_ref[...]
    x = jnp.dot(el, W1c[...], preferred_element_type=_F32) + G_ref[...] + b1[...]
    h1 = jnp.maximum(x, 0.0)
    h2 = jnp.maximum(jnp.dot(h1, W2[...], preferred_element_type=_F32) + b2[...], 0.0)
    out = _ln(h2, g[...], bb[...])
    nel[...] = out
    elo[...] = el + out


def _node_step_body(nl_ref, a0_ref, a1_ref, Wa, Wb, b1, W2, b2, g, bb,
                    WnA, WnB, nlo, A, B):
    nl = nl_ref[...]
    agg = a0_ref[...] + a1_ref[...]
    x = (jnp.dot(nl, Wa[...], preferred_element_type=_F32)
         + jnp.dot(agg, Wb[...], preferred_element_type=_F32) + b1[...])
    h1 = jnp.maximum(x, 0.0)
    h2 = jnp.maximum(jnp.dot(h1, W2[...], preferred_element_type=_F32) + b2[...], 0.0)
    out = nl + _ln(h2, g[...], bb[...])
    nlo[...] = out
    A[...] = jnp.dot(out, WnA[...], preferred_element_type=_F32)
    B[...] = jnp.dot(out, WnB[...], preferred_element_type=_F32)


def _node_last_body(nl_ref, a0_ref, a1_ref, Wa, Wb, b1, W2, b2, g, bb,
                    dW1, db1, dW2, db2, dtp, dec):
    nl = nl_ref[...]
    agg = a0_ref[...] + a1_ref[...]
    x = (jnp.dot(nl, Wa[...], preferred_element_type=_F32)
         + jnp.dot(agg, Wb[...], preferred_element_type=_F32) + b1[...])
    h1 = jnp.maximum(x, 0.0)
    h2 = jnp.maximum(jnp.dot(h1, W2[...], preferred_element_type=_F32) + b2[...], 0.0)
    out = nl + _ln(h2, g[...], bb[...])
    h = jnp.dot(out, dW1[...], preferred_element_type=_F32) + db1[...]
    h = h * jax.nn.sigmoid(h)
    dec[...] = (jnp.dot(h, dW2[...], preferred_element_type=_F32) + db2[...]) * dtp[...]


def _row_spec(blk, w):
    return pl.BlockSpec((blk, w), lambda i: (i, 0))


def _full_spec(shape):
    nd = len(shape)
    return pl.BlockSpec(shape, lambda i: (0,) * nd)


def _tc_call(body, grid, in_arrays, blocked, out_shapes, out_blocked):
    in_specs = []
    for a, b in zip(in_arrays, blocked):
        in_specs.append(_row_spec(*b) if b is not None else _full_spec(a.shape))
    out_specs = [_row_spec(*b) if b is not None else _full_spec(s.shape)
                 for s, b in zip(out_shapes, out_blocked)]
    return pl.pallas_call(
        body,
        grid=(grid,),
        in_specs=in_specs,
        out_specs=out_specs if len(out_specs) > 1 else out_specs[0],
        out_shape=out_shapes if len(out_shapes) > 1 else out_shapes[0],
    )(*in_arrays)


def _r1(v):
    return v.reshape(1, L)


def kernel(mesh_pos, node_type, u, load, senders, receivers,
           ne_W1, ne_b1, ne_W2, ne_b2, ne_g, ne_bb,
           ee_W1, ee_b1, ee_W2, ee_b2, ee_g, ee_bb,
           be_W1, be_b1, be_W2, be_b2, be_g, be_bb,
           bn_W1, bn_b1, bn_W2, bn_b2, bn_g, bn_bb,
           dec_W1, dec_b1, dec_W2, dec_b2):
    mesh_pos = mesh_pos[0]
    node_type = node_type[0]
    u = u[0]
    load = load[0]

    pad_e = E_PAD - E
    s_pad = jnp.concatenate([senders.astype(jnp.int32),
                             jnp.full((pad_e,), N, jnp.int32)])
    r_pad = jnp.concatenate([receivers.astype(jnp.int32),
                             jnp.full((pad_e,), N, jnp.int32)])
    s_w = s_pad.reshape(NWORK, CH_PER_W, CHUNK)
    r_w = r_pad.reshape(NWORK, CH_PER_W, CHUNK)

    T = jnp.zeros((N_PAD, L), _F32)
    T = T.at[:N, 0:3].set(mesh_pos).at[:N, 3:6].set(u)
    Tn = -T

    diff = _sc_gather_add(T, Tn, s_w, r_w, D=L)

    W1d = jnp.zeros((L, L), _F32).at[0:6, :].set(ee_W1[0:6, :])
    w1n = ee_W1[6:7, :]
    el = _tc_call(
        _edge_encode_body, E_PAD // BLK_E,
        [diff, W1d, w1n, _r1(ee_b1), ee_W2, _r1(ee_b2), _r1(ee_g), _r1(ee_bb)],
        [(BLK_E, L), None, None, None, None, None, None, None],
        [jax.ShapeDtypeStruct((E_PAD, L), _F32)], [(BLK_E, L)])

    nf = jnp.zeros((N_PAD, 16), _F32)
    nf = nf.at[:N, 0:3].set(u).at[:N, 3:6].set(load).at[:N, 6:15].set(node_type)
    neW1p = jnp.zeros((16, L), _F32).at[0:15, :].set(ne_W1)
    nl, A, B = _tc_call(
        _node_encode_body, N_PAD // BLK_N,
        [nf, neW1p, _r1(ne_b1), ne_W2, _r1(ne_b2), _r1(ne_g), _r1(ne_bb),
         be_W1[0, 0:L, :], be_W1[0, L:2 * L, :]],
        [(BLK_N, 16)] + [None] * 8,
        [jax.ShapeDtypeStruct((N_PAD, L), _F32)] * 3,
        [(BLK_N, L)] * 3)

    zeros_n = jnp.zeros((N_PAD, L), _F32)

    for i in range(STEPS):
        G = _sc_gather_add(A, B, s_w, r_w, D=L)
        el, new_el = _tc_call(
            _edge_step_body, E_PAD // BLK_E,
            [el, G, be_W1[i, 2 * L:3 * L, :], _r1(be_b1[i]), be_W2[i],
             _r1(be_b2[i]), _r1(be_g[i]), _r1(be_bb[i])],
            [(BLK_E, L), (BLK_E, L), None, None, None, None, None, None],
            [jax.ShapeDtypeStruct((E_PAD, L), _F32)] * 2,
            [(BLK_E, L)] * 2)

        parts = _sc_scatter_add(new_el, r_w, zeros_n)
        ag0, ag1 = parts[0], parts[1]

        if i < STEPS - 1:
            nl, A, B = _tc_call(
                _node_step_body, N_PAD // BLK_N,
                [nl, ag0, ag1, bn_W1[i, 0:L, :], bn_W1[i, L:2 * L, :],
                 _r1(bn_b1[i]), bn_W2[i], _r1(bn_b2[i]), _r1(bn_g[i]),
                 _r1(bn_bb[i]), be_W1[i + 1, 0:L, :], be_W1[i + 1, L:2 * L, :]],
                [(BLK_N, L)] * 3 + [None] * 9,
                [jax.ShapeDtypeStruct((N_PAD, L), _F32)] * 3,
                [(BLK_N, L)] * 3)
        else:
            dW1p = jnp.zeros((L, L), _F32).at[:, 0:8].set(dec_W1)
            db1p = jnp.zeros((1, L), _F32).at[0, 0:8].set(dec_b1)
            dW2p = jnp.zeros((L, L), _F32).at[0:8, 0:TD * TW].set(dec_W2)
            db2p = jnp.zeros((1, L), _F32).at[0, 0:TD * TW].set(dec_b2)
            dt = jnp.repeat(jnp.arange(1, TW + 1, dtype=_F32), TD)
            dtp = jnp.zeros((1, L), _F32).at[0, 0:TD * TW].set(dt)
            dec = _tc_call(
                _node_last_body, N_PAD // BLK_N,
                [nl, ag0, ag1, bn_W1[i, 0:L, :], bn_W1[i, L:2 * L, :],
                 _r1(bn_b1[i]), bn_W2[i], _r1(bn_b2[i]), _r1(bn_g[i]),
                 _r1(bn_bb[i]), dW1p, db1p, dW2p, db2p, dtp],
                [(BLK_N, L)] * 3 + [None] * 12,
                [jax.ShapeDtypeStruct((N_PAD, L), _F32)], [(BLK_N, L)])

    return dec[:N, 0:TD * TW].reshape(N, TW, TD).transpose(1, 0, 2)
